# Initial kernel scaffold; baseline (speedup 1.0000x reference)
import jax
import jax.numpy as jnp
from jax.experimental import pallas as pl


def kernel(vertices, faces):
    raise NotImplementedError("write your pallas kernel here")


# speedup vs baseline: 9.3887x; 9.3887x over previous
import functools

import jax
import jax.numpy as jnp
from jax import lax
from jax.experimental import pallas as pl
from jax.experimental.pallas import tpu as pltpu
from jax.experimental.pallas import tpu_sc as plsc

NC = 2
NS = 16
NW = NC * NS
L = 16


@functools.lru_cache(maxsize=None)
def _face_normals_sc(NP):
    CH = NP // NW
    mesh = plsc.VectorSubcoreMesh(core_axis_name="c", subcore_axis_name="s")
    out_t = [jax.ShapeDtypeStruct((NP,), jnp.float32)] * 3
    scratch = (
        [pltpu.VMEM((CH,), jnp.int32)] * 3
        + [pltpu.VMEM((CH,), jnp.float32)] * 12
        + [pltpu.SemaphoreType.DMA]
    )

    @functools.partial(
        pl.kernel, mesh=mesh, out_type=out_t, scratch_types=scratch,
        compiler_params=pltpu.CompilerParams(needs_layout_passes=False))
    def k(vx, vy, vz, f0, f1, f2, onx, ony, onz,
          i0, i1, i2, x0, y0, z0, x1, y1, z1, x2, y2, z2, ox, oy, oz, sem):
        wid = lax.axis_index("s") * NC + lax.axis_index("c")
        base = wid * CH
        pltpu.sync_copy(f0.at[pl.ds(base, CH)], i0)
        pltpu.sync_copy(f1.at[pl.ds(base, CH)], i1)
        pltpu.sync_copy(f2.at[pl.ds(base, CH)], i2)
        cps = [
            pltpu.async_copy(vx.at[i0], x0, sem),
            pltpu.async_copy(vy.at[i0], y0, sem),
            pltpu.async_copy(vz.at[i0], z0, sem),
            pltpu.async_copy(vx.at[i1], x1, sem),
            pltpu.async_copy(vy.at[i1], y1, sem),
            pltpu.async_copy(vz.at[i1], z1, sem),
            pltpu.async_copy(vx.at[i2], x2, sem),
            pltpu.async_copy(vy.at[i2], y2, sem),
            pltpu.async_copy(vz.at[i2], z2, sem),
        ]
        for c in cps:
            c.wait()

        def step(i, carry):
            s = pl.ds(i * L, L)
            ax0 = x0[s]; ay0 = y0[s]; az0 = z0[s]
            ax1 = x1[s]; ay1 = y1[s]; az1 = z1[s]
            ax2 = x2[s]; ay2 = y2[s]; az2 = z2[s]
            e1x = ax0 - ax1; e1y = ay0 - ay1; e1z = az0 - az1
            e2x = ax2 - ax1; e2y = ay2 - ay1; e2z = az2 - az1
            nx = e2y * e1z - e2z * e1y
            ny = e2z * e1x - e2x * e1z
            nz = e2x * e1y - e2y * e1x
            nn = nx * nx + ny * ny + nz * nz
            ii = jnp.int32(0x5F3759DF) - (plsc.bitcast(nn, jnp.int32) >> 1)
            r = plsc.bitcast(ii, jnp.float32)
            h = nn * jnp.float32(0.5)
            r = r * (jnp.float32(1.5) - (h * r) * r)
            r = r * (jnp.float32(1.5) - (h * r) * r)
            ox[s] = nx * r
            oy[s] = ny * r
            oz[s] = nz * r
            return carry

        lax.fori_loop(0, CH // L, step, 0, unroll=4)

        pltpu.sync_copy(ox, onx.at[pl.ds(base, CH)])
        pltpu.sync_copy(oy, ony.at[pl.ds(base, CH)])
        pltpu.sync_copy(oz, onz.at[pl.ds(base, CH)])

    return k


def kernel(vertices, faces):
    fi = faces.astype(jnp.int32)
    N = fi.shape[0]
    NP = -(-N // (NW * 128)) * (NW * 128)
    pad = NP - N
    f0 = jnp.pad(fi[:, 0], (0, pad))
    f1 = jnp.pad(fi[:, 1], (0, pad))
    f2 = jnp.pad(fi[:, 2], (0, pad))
    onx, ony, onz = _face_normals_sc(NP)(
        vertices[:, 0], vertices[:, 1], vertices[:, 2], f0, f1, f2)
    return jnp.stack([onx[:N], ony[:N], onz[:N]], axis=-1)
